# Initial kernel scaffold; baseline (speedup 1.0000x reference)
import jax
import jax.numpy as jnp
from jax.experimental import pallas as pl


def kernel(x, params):
    raise NotImplementedError("write your pallas kernel here")


# speedup vs baseline: 1.1612x; 1.1612x over previous
import jax
import jax.numpy as jnp
from jax.experimental import pallas as pl
from jax.experimental.pallas import tpu as pltpu

DIM = 512
PS = 16
IMG = 224
NH = 2
DH = 64
DEPTH = 6
E = 6
NC = 1000
HP = IMG // PS
NP = HP * HP
PD = PS * PS * 3
NPAD = 256
ROWS = 4 * NPAD
B = 4

_DT = jnp.bfloat16
_INTERPRET = False


def _mm(a, b):
    return jax.lax.dot(a.astype(_DT), b.astype(_DT),
                       preferred_element_type=jnp.float32)


def _mm_t(a, b):
    return jax.lax.dot_general(a.astype(_DT), b.astype(_DT),
                               (((1,), (1,)), ((), ())),
                               preferred_element_type=jnp.float32)


def _ln(v, g, b):
    m = jnp.mean(v, axis=-1, keepdims=True)
    c = v - m
    var = jnp.mean(c * c, axis=-1, keepdims=True)
    return c / jnp.sqrt(var + 1e-5) * g + b


def _top1(logits):
    mx = jnp.max(logits, axis=-1, keepdims=True)
    lane = jax.lax.broadcasted_iota(jnp.int32, logits.shape, 1)
    first = jnp.min(jnp.where(logits == mx, lane, logits.shape[-1]),
                    axis=-1, keepdims=True)
    return mx, first


def _body(pp_ref, cls_ref, pos_ref,
          pw_ref, pbias_ref, pg1_ref, pb1_ref, pg2_ref, pb2_ref,
          ln1g_ref, ln1b_ref, ln2g_ref, ln2b_ref,
          wq_ref, bq_ref, wk_ref, bk_ref,
          wsv_ref, wso_ref, wv_ref, wo_ref,
          rw_ref, rb_ref, ew_ref, eb_ref,
          ng_ref, nb_ref, hw_ref, hb_ref,
          out_ref, x_scr):
    l = pl.program_id(0)

    @pl.when(l == 0)
    def _patch():
        pe = _ln(pp_ref[...], pg1_ref[...], pb1_ref[...])
        emb = _mm(pe, pw_ref[...]) + pbias_ref[...]
        emb = _ln(emb, pg2_ref[...], pb2_ref[...])
        pos = pos_ref[...]
        for bi in range(B):
            base = bi * NPAD
            x_scr[base:base + NPAD, :] = emb[base:base + NPAD] + pos
            x_scr[base:base + 1, :] = cls_ref[...] + pos[0:1]
            x_scr[base + 197:base + NPAD, :] = jnp.zeros((NPAD - 197, DIM),
                                                         jnp.float32)

    x = x_scr[...]

    xn = _ln(x, ln1g_ref[0], ln1b_ref[0])

    qs = [_mm(xn, wq_ref[0, h]) + bq_ref[0, h] for h in range(NH)]
    ks = [_mm(xn, wk_ref[0, h]) + bk_ref[0, h] for h in range(NH)]

    vs = []
    for h in range(NH):
        svl = _mm(xn, wsv_ref[0, h])
        mx, first = _top1(svl)
        val = jax.nn.sigmoid(mx)
        vh = jnp.zeros((ROWS, DH), jnp.float32)
        for e in range(E):
            coef = jnp.where(first == e, val, 0.0)
            vh = vh + coef * _mm(xn, wv_ref[0, h, e])
        vs.append(vh)

    col = jax.lax.broadcasted_iota(jnp.int32, (NPAD, NPAD), 1)
    neg = jnp.float32(-1e30)
    ohs = []
    for h in range(NH):
        parts = []
        for bi in range(B):
            sl = slice(bi * NPAD, (bi + 1) * NPAD)
            att = _mm_t(qs[h][sl], ks[h][sl]) / jnp.sqrt(jnp.float32(DH))
            att = jnp.where(col < 197, att, neg)
            amx = jnp.max(att, axis=-1, keepdims=True)
            ex = jnp.exp(att - amx)
            a = ex / jnp.sum(ex, axis=-1, keepdims=True)
            parts.append(_mm(a, vs[h][sl]))
        ohs.append(jnp.concatenate(parts, axis=0))

    attn_out = jnp.zeros((ROWS, DIM), jnp.float32)
    for h in range(NH):
        sol = _mm(xn, wso_ref[0, h])
        mx, first = _top1(sol)
        val = jax.nn.sigmoid(mx)
        for e in range(E):
            coef = jnp.where(first == e, val, 0.0)
            attn_out = attn_out + coef * _mm(ohs[h], wo_ref[0, h, e])

    x = x + attn_out

    xn2 = _ln(x, ln2g_ref[0], ln2b_ref[0])
    rl = _mm(xn2, rw_ref[0]) + rb_ref[0]
    mx, first = _top1(rl)
    gval = 1.0 / jnp.sum(jnp.exp(rl - mx), axis=-1, keepdims=True)
    y = jnp.zeros((ROWS, DIM), jnp.float32)
    for e in range(E):
        coef = jnp.where(first == e, gval, 0.0)
        y = y + coef * (_mm(xn2, ew_ref[0, e]) + eb_ref[0, e])
    x = x + y

    x_scr[...] = x

    @pl.when(l == DEPTH - 1)
    def _head():
        rows = jnp.concatenate(
            [x[bi * NPAD:bi * NPAD + 1] for bi in range(B)], axis=0)
        hn = _ln(rows, ng_ref[...], nb_ref[...])
        out_ref[...] = _mm(hn, hw_ref[...]) + hb_ref[...]


def kernel(x, params):
    p = params
    L = DEPTH

    patches = x.reshape(B, 3, HP, PS, HP, PS).transpose(0, 2, 4, 3, 5, 1)
    patches = patches.reshape(B, NP, PD)
    pp = jnp.pad(patches, ((0, 0), (1, NPAD - 1 - NP), (0, 0)))
    pp = pp.reshape(ROWS, PD)

    pos = jnp.pad(p['pos_enc'][0], ((0, NPAD - 197), (0, 0)))
    cls = p['class_token'].reshape(1, DIM)

    wq = p['Wq'].reshape(L, DIM, NH, DH).transpose(0, 2, 1, 3)
    bq = p['bq'].reshape(L, NH, 1, DH)
    wk = p['Wk'].reshape(L, DIM, NH, DH).transpose(0, 2, 1, 3)
    bk = p['bk'].reshape(L, NH, 1, DH)
    wsv = p['Wsv'].reshape(L, DIM, NH, E).transpose(0, 2, 1, 3)
    wso = p['Wso'].reshape(L, DIM, NH, E).transpose(0, 2, 1, 3)

    inputs = [
        pp, cls, pos,
        p['patch_W'], p['patch_b'].reshape(1, DIM),
        p['patch_ln1_g'].reshape(1, PD), p['patch_ln1_b'].reshape(1, PD),
        p['patch_ln2_g'].reshape(1, DIM), p['patch_ln2_b'].reshape(1, DIM),
        p['ln1_g'].reshape(L, 1, DIM), p['ln1_b'].reshape(L, 1, DIM),
        p['ln2_g'].reshape(L, 1, DIM), p['ln2_b'].reshape(L, 1, DIM),
        wq, bq, wk, bk, wsv, wso,
        p['Wv'], p['Wo'],
        p['router_W'], p['router_b'].reshape(L, 1, E),
        p['expert_W'], p['expert_b'].reshape(L, E, 1, DIM),
        p['norm_g'].reshape(1, DIM), p['norm_b'].reshape(1, DIM),
        p['head_W'], p['head_b'].reshape(1, NC),
    ]

    def stacked(a):
        shp = a.shape
        return pl.BlockSpec((1,) + shp[1:],
                            lambda l, n=len(shp): (l,) + (0,) * (n - 1))

    def const(a):
        shp = a.shape
        return pl.BlockSpec(shp, lambda l, n=len(shp): (0,) * n)

    per_layer = {9, 10, 11, 12, 13, 14, 15, 16, 17, 18, 19, 20, 21, 22, 23, 24}
    in_specs = [stacked(a) if i in per_layer else const(a)
                for i, a in enumerate(inputs)]

    out = pl.pallas_call(
        _body,
        grid=(DEPTH,),
        in_specs=in_specs,
        out_specs=pl.BlockSpec((B, NC), lambda l: (0, 0)),
        out_shape=jax.ShapeDtypeStruct((B, NC), jnp.float32),
        scratch_shapes=[pltpu.VMEM((ROWS, DIM), jnp.float32)],
        compiler_params=pltpu.CompilerParams(
            dimension_semantics=("arbitrary",)),
        interpret=_INTERPRET,
    )(*inputs)
    return out
